# baseline (speedup 1.0000x reference)
import functools

import jax
import jax.numpy as jnp
from jax import lax
from jax.experimental import pallas as pl
from jax.experimental.pallas import tpu as pltpu
from jax.experimental.pallas import tpu_sc as plsc

_M, _H = 8192, 1024
_NC, _NS = 2, 16
_NW = _NC * _NS
_R = 512
_RPW = _R // _NW


def _sc_head_body(table, out, buf, isem, osem):
    c = lax.axis_index("c")
    s = lax.axis_index("s")
    wid = s * _NC + c
    base = wid * _RPW
    pltpu.make_async_copy(table.at[pl.ds(base, _RPW)], buf, isem).start()
    pltpu.make_async_copy(table.at[pl.ds(base, _RPW)], buf, isem).wait()
    pltpu.make_async_copy(buf, out.at[pl.ds(base, _RPW)], osem).start()
    pltpu.make_async_copy(buf, out.at[pl.ds(base, _RPW)], osem).wait()


@functools.partial(
    pl.kernel,
    mesh=plsc.VectorSubcoreMesh(core_axis_name="c", subcore_axis_name="s"),
    out_type=jax.ShapeDtypeStruct((_M, _H), jnp.float32),
    scratch_types=[
        pltpu.VMEM((_RPW, _H), jnp.float32),
        pltpu.SemaphoreType.DMA,
        pltpu.SemaphoreType.DMA,
    ],
)
def _sc_head(table, out, buf, isem, osem):
    _sc_head_body(table, out, buf, isem, osem)


_BOUNDS = (_R, _R + (_M - _R) // 2, _M)


def _tc_tail_kernel(table_ref, head_ref, out_ref, vbuf, in_sem, out_sem):
    del head_ref
    n = len(_BOUNDS) - 1
    loads = [
        pltpu.make_async_copy(
            table_ref.at[pl.ds(_BOUNDS[k], _BOUNDS[k + 1] - _BOUNDS[k])],
            vbuf.at[pl.ds(_BOUNDS[k] - _R, _BOUNDS[k + 1] - _BOUNDS[k])],
            in_sem.at[k],
        )
        for k in range(n)
    ]
    stores = [
        pltpu.make_async_copy(
            vbuf.at[pl.ds(_BOUNDS[k] - _R, _BOUNDS[k + 1] - _BOUNDS[k])],
            out_ref.at[pl.ds(_BOUNDS[k], _BOUNDS[k + 1] - _BOUNDS[k])],
            out_sem.at[k],
        )
        for k in range(n)
    ]
    for k in range(n):
        loads[k].start()
    for k in range(n):
        loads[k].wait()
        stores[k].start()
    for k in range(n):
        stores[k].wait()


def kernel(seq_len, position_embedding):
    del seq_len
    M, H = position_embedding.shape
    head = _sc_head(position_embedding)
    n = len(_BOUNDS) - 1
    return pl.pallas_call(
        _tc_tail_kernel,
        in_specs=[
            pl.BlockSpec(memory_space=pltpu.MemorySpace.HBM),
            pl.BlockSpec(memory_space=pltpu.MemorySpace.HBM),
        ],
        out_specs=pl.BlockSpec(memory_space=pltpu.MemorySpace.HBM),
        out_shape=jax.ShapeDtypeStruct((M, H), position_embedding.dtype),
        scratch_shapes=[
            pltpu.VMEM((M - _R, H), position_embedding.dtype),
            pltpu.SemaphoreType.DMA((n,)),
            pltpu.SemaphoreType.DMA((n,)),
        ],
        input_output_aliases={1: 0},
    )(position_embedding, head)


# speedup vs baseline: 1.9440x; 1.9440x over previous
import jax
import jax.numpy as jnp
from jax.experimental import pallas as pl
from jax.experimental.pallas import tpu as pltpu

_BOUNDS = (0, 4096, 8192)


def _dma_copy_kernel(in_ref, out_ref, vbuf, in_sem, out_sem):
    n = len(_BOUNDS) - 1
    loads = [
        pltpu.make_async_copy(
            in_ref.at[pl.ds(_BOUNDS[k], _BOUNDS[k + 1] - _BOUNDS[k])],
            vbuf.at[pl.ds(_BOUNDS[k], _BOUNDS[k + 1] - _BOUNDS[k])],
            in_sem.at[k],
        )
        for k in range(n)
    ]
    stores = [
        pltpu.make_async_copy(
            vbuf.at[pl.ds(_BOUNDS[k], _BOUNDS[k + 1] - _BOUNDS[k])],
            out_ref.at[pl.ds(_BOUNDS[k], _BOUNDS[k + 1] - _BOUNDS[k])],
            out_sem.at[k],
        )
        for k in range(n)
    ]
    for k in range(n):
        loads[k].start()
    for k in range(n):
        loads[k].wait()
        stores[k].start()
    for k in range(n):
        stores[k].wait()


def kernel(seq_len, position_embedding):
    del seq_len
    M, H = position_embedding.shape
    n = len(_BOUNDS) - 1
    return pl.pallas_call(
        _dma_copy_kernel,
        in_specs=[pl.BlockSpec(memory_space=pltpu.MemorySpace.HBM)],
        out_specs=pl.BlockSpec(memory_space=pltpu.MemorySpace.HBM),
        out_shape=jax.ShapeDtypeStruct((M, H), position_embedding.dtype),
        scratch_shapes=[
            pltpu.VMEM((M, H), position_embedding.dtype),
            pltpu.SemaphoreType.DMA((n,)),
            pltpu.SemaphoreType.DMA((n,)),
        ],
    )(position_embedding)
